# baseline (speedup 1.0000x reference)
import functools

import jax
import jax.numpy as jnp
from jax import lax
from jax.experimental import pallas as pl
from jax.experimental.pallas import tpu as pltpu
from jax.experimental.pallas import tpu_sc as plsc

N = 50000
E = 800000
G = 128

NC = 2
NS = 16
CHD = 1000
CHS = 2000
NP = 50176
RPS = NP // NS
DRN = 784
LW = 16

_mesh = plsc.VectorSubcoreMesh(core_axis_name="c", subcore_axis_name="s")
_UNTILED = pltpu.CompilerParams(use_tc_tiling_on_sc=False)

NB = 1000


@functools.partial(
    pl.kernel, mesh=_mesh,
    out_type=jax.ShapeDtypeStruct((2, NC, NP, LW), jnp.float32),
    scratch_types=[
        pltpu.VMEM((CHD,), jnp.int32),
        pltpu.VMEM((CHD, LW), jnp.float32),
        pltpu.VMEM((DRN, LW), jnp.float32),
        pltpu.VMEM_SHARED((NP, LW), jnp.float32),
    ],
    compiler_params=_UNTILED,
)
def _sc_degree(dst1_hbm, dst2_hbm, out_hbm, didx_v, ones_v, z_v, acc_sh):
    cid = lax.axis_index("c")
    sid = lax.axis_index("s")
    ones_v[...] = jnp.ones_like(ones_v)
    ecore = E // NC
    esub = ecore // NS
    for g, dst_hbm in ((0, dst1_hbm), (1, dst2_hbm)):
        z_v[...] = jnp.zeros_like(z_v)
        for k in range(RPS // DRN):
            pltpu.sync_copy(z_v, acc_sh.at[pl.ds(sid * RPS + k * DRN, DRN)])
        plsc.subcore_barrier()

        def _chunk(k, carry, dst_hbm=dst_hbm):
            base = cid * ecore + sid * esub + k * CHD
            pltpu.sync_copy(dst_hbm.at[pl.ds(base, CHD)], didx_v)
            pltpu.sync_copy(ones_v, acc_sh.at[didx_v], add=True)
            return carry

        lax.fori_loop(0, esub // CHD, _chunk, 0)
        plsc.subcore_barrier()
        for k in range(RPS // DRN):
            pltpu.sync_copy(acc_sh.at[pl.ds(sid * RPS + k * DRN, DRN)], z_v)
            pltpu.sync_copy(z_v, out_hbm.at[g].at[cid].at[pl.ds(sid * RPS + k * DRN, DRN)])
        plsc.subcore_barrier()


def _make_sc_spmm(SL):
    @functools.partial(
        pl.kernel, mesh=_mesh,
        out_type=jax.ShapeDtypeStruct((2, SL, NP, LW), jnp.float32),
        scratch_types=[
            pltpu.VMEM((CHS,), jnp.int32),
            pltpu.VMEM((CHS,), jnp.int32),
            pltpu.VMEM((CHS,), jnp.int32),
            pltpu.VMEM((CHS, LW), jnp.float32),
            pltpu.VMEM((DRN, LW), jnp.float32),
            pltpu.VMEM_SHARED((NP, LW), jnp.float32),
            pltpu.SemaphoreType.DMA,
        ],
        compiler_params=_UNTILED,
    )
    def _sc_spmm(h41_hbm, src1_hbm, dst1_hbm, h42_hbm, src2_hbm, dst2_hbm,
                 out_hbm, sidx_v, didx_v, gidx_v, rows_v, z_v, acc_sh, sem):
        cid = lax.axis_index("c")
        sid = lax.axis_index("s")
        esub = E // NS
        slc = SL // NC
        for g, h4_hbm, src_hbm, dst_hbm in (
            (0, h41_hbm, src1_hbm, dst1_hbm),
            (1, h42_hbm, src2_hbm, dst2_hbm),
        ):
            def _pass(p, carry, h4_hbm=h4_hbm, src_hbm=src_hbm,
                      dst_hbm=dst_hbm, g=g):
                c = cid * slc + p
                z_v[...] = jnp.zeros_like(z_v)
                for k in range(RPS // DRN):
                    pltpu.sync_copy(z_v, acc_sh.at[pl.ds(sid * RPS + k * DRN, DRN)])
                plsc.subcore_barrier()

                def _chunk(k, carry2):
                    base = sid * esub + k * CHS
                    pltpu.sync_copy(src_hbm.at[pl.ds(base, CHS)], sidx_v)
                    pltpu.sync_copy(dst_hbm.at[pl.ds(base, CHS)], didx_v)
                    gidx_v[...] = sidx_v[...] * SL + c
                    pltpu.async_copy(h4_hbm.at[gidx_v], rows_v, sem).wait()
                    pltpu.sync_copy(rows_v, acc_sh.at[didx_v], add=True)
                    return carry2

                lax.fori_loop(0, esub // CHS, _chunk, 0)
                plsc.subcore_barrier()
                for k in range(RPS // DRN):
                    pltpu.sync_copy(acc_sh.at[pl.ds(sid * RPS + k * DRN, DRN)], z_v)
                    pltpu.sync_copy(z_v, out_hbm.at[g].at[c].at[pl.ds(sid * RPS + k * DRN, DRN)])
                plsc.subcore_barrier()
                return carry

            lax.fori_loop(0, slc, _pass, 0)

    return _sc_spmm


_sc_spmm8 = _make_sc_spmm(8)
_sc_spmm16 = _make_sc_spmm(16)


def _dinv_from_deg(deg_ref):
    deg = deg_ref[0][:, 0:1] + deg_ref[1][:, 0:1] + 1.0
    return lax.rsqrt(jnp.maximum(deg, 1.0))


def _l1_body(x_ref, W_ref, deg_ref, h_ref, hp_ref):
    dinv = _dinv_from_deg(deg_ref)
    h = jnp.dot(x_ref[...], W_ref[...], preferred_element_type=jnp.float32)
    h_ref[...] = h
    hp_ref[...] = h * dinv


def _mid_body(SL_prev, S_ref, hprev_ref, deg_ref, W_ref, b_ref, h_ref, hp_ref):
    dinv = _dinv_from_deg(deg_ref)
    S = jnp.concatenate([S_ref[c] for c in range(SL_prev)], axis=1)
    x = jax.nn.relu(dinv * S + (dinv * dinv) * hprev_ref[...] + b_ref[...])
    h = jnp.dot(x, W_ref[...], preferred_element_type=jnp.float32)
    h_ref[...] = h
    hp_ref[...] = h * dinv


def _fin_body(S_ref, hprev_ref, deg_ref, b_ref, x_ref):
    dinv = _dinv_from_deg(deg_ref)
    S = jnp.concatenate([S_ref[c] for c in range(16)], axis=1)
    x_ref[...] = jax.nn.relu(dinv * S + (dinv * dinv) * hprev_ref[...] + b_ref[...])


def _deg_spec():
    return pl.BlockSpec((NC, NB, LW), lambda i: (0, i, 0))


def _full(shape):
    return pl.BlockSpec(shape, lambda i: tuple(0 for _ in shape))


def _tc_layer1(x0, W, deg):
    return pl.pallas_call(
        _l1_body,
        grid=(N // NB,),
        in_specs=[
            pl.BlockSpec((NB, x0.shape[1]), lambda i: (i, 0)),
            _full(W.shape),
            _deg_spec(),
        ],
        out_specs=[
            pl.BlockSpec((NB, W.shape[1]), lambda i: (i, 0)),
            pl.BlockSpec((NB, W.shape[1]), lambda i: (i, 0)),
        ],
        out_shape=[
            jax.ShapeDtypeStruct((N, W.shape[1]), jnp.float32),
            jax.ShapeDtypeStruct((N, W.shape[1]), jnp.float32),
        ],
    )(x0, W, deg)


def _tc_layer_mid(S, hprev, deg, W, b):
    SLp = S.shape[0]
    return pl.pallas_call(
        functools.partial(_mid_body, SLp),
        grid=(N // NB,),
        in_specs=[
            pl.BlockSpec((SLp, NB, LW), lambda i: (0, i, 0)),
            pl.BlockSpec((NB, hprev.shape[1]), lambda i: (i, 0)),
            _deg_spec(),
            _full(W.shape),
            _full((1, SLp * LW)),
        ],
        out_specs=[
            pl.BlockSpec((NB, W.shape[1]), lambda i: (i, 0)),
            pl.BlockSpec((NB, W.shape[1]), lambda i: (i, 0)),
        ],
        out_shape=[
            jax.ShapeDtypeStruct((N, W.shape[1]), jnp.float32),
            jax.ShapeDtypeStruct((N, W.shape[1]), jnp.float32),
        ],
    )(S, hprev, deg, W, b.reshape(1, -1))


def _tc_layer_fin(S, hprev, deg, b):
    return pl.pallas_call(
        _fin_body,
        grid=(N // NB,),
        in_specs=[
            pl.BlockSpec((16, NB, LW), lambda i: (0, i, 0)),
            pl.BlockSpec((NB, 256), lambda i: (i, 0)),
            _deg_spec(),
            _full((1, 256)),
        ],
        out_specs=pl.BlockSpec((NB, 256), lambda i: (i, 0)),
        out_shape=jax.ShapeDtypeStruct((N, 256), jnp.float32),
    )(S, hprev, deg, b.reshape(1, -1))


def _head_body(x12_ref, fc1_W_ref, fc1_b_ref, fc2_W_ref, fc2_b_ref,
               fc3_W_ref, fc3_b_ref, out_ref):
    x = x12_ref[...]
    h1 = jax.nn.relu(jnp.dot(x, fc1_W_ref[...], preferred_element_type=jnp.float32) + fc1_b_ref[...])
    h2 = jax.nn.relu(jnp.dot(h1, fc2_W_ref[...], preferred_element_type=jnp.float32) + fc2_b_ref[...])
    h3 = jnp.tanh(jnp.dot(h2, fc3_W_ref[...], preferred_element_type=jnp.float32) + fc3_b_ref[...])
    out_ref[...] = h3


def _head(x, fc1_W, fc1_b, fc2_W, fc2_b, fc3_W, fc3_b):
    return pl.pallas_call(
        _head_body,
        out_shape=jax.ShapeDtypeStruct((G, 128), jnp.float32),
    )(x, fc1_W, fc1_b.reshape(1, -1), fc2_W, fc2_b.reshape(1, -1),
      jnp.pad(fc3_W, ((0, 0), (0, 127))), jnp.pad(fc3_b, (0, 127)).reshape(1, -1))


def _pool(xf, batch):
    cnt = jax.ops.segment_sum(jnp.ones((N,), jnp.float32), batch, num_segments=G)
    gmp = jax.ops.segment_max(xf, batch, num_segments=G)
    gmp = jnp.where(cnt[:, None] > 0, gmp, 0.0)
    gap = jax.ops.segment_sum(xf, batch, num_segments=G) / jnp.maximum(cnt, 1.0)[:, None]
    return jnp.concatenate([gmp, gap], axis=1)


def kernel(graph1_x, graph1_edge_index, graph1_batch, graph2_x, graph2_edge_index, graph2_batch, W1, b1, W2, b2, W3, b3, W4, b4, fc1_W, fc1_b, fc2_W, fc2_b, fc3_W, fc3_b):
    src1, dst1 = graph1_edge_index[0], graph1_edge_index[1]
    src2, dst2 = graph2_edge_index[0], graph2_edge_index[1]
    degs = _sc_degree(dst1, dst2)
    dega, degb = degs[0], degs[1]

    xa = jnp.pad(graph1_x, ((0, 0), (0, 128 - graph1_x.shape[1])))
    xb = jnp.pad(graph2_x, ((0, 0), (0, 128 - graph2_x.shape[1])))
    W1p = jnp.pad(W1, ((0, 128 - W1.shape[0]), (0, 0)))

    h1a, hp1a = _tc_layer1(xa, W1p, dega)
    h1b, hp1b = _tc_layer1(xb, W1p, degb)
    S1 = _sc_spmm8(hp1a.reshape(N * 8, LW), src1, dst1,
                   hp1b.reshape(N * 8, LW), src2, dst2)
    h2a, hp2a = _tc_layer_mid(S1[0], h1a, dega, W2, b1)
    h2b, hp2b = _tc_layer_mid(S1[1], h1b, degb, W2, b1)
    S2 = _sc_spmm16(hp2a.reshape(N * 16, LW), src1, dst1,
                    hp2b.reshape(N * 16, LW), src2, dst2)
    h3a, hp3a = _tc_layer_mid(S2[0], h2a, dega, W3, b2)
    h3b, hp3b = _tc_layer_mid(S2[1], h2b, degb, W3, b2)
    S3 = _sc_spmm16(hp3a.reshape(N * 16, LW), src1, dst1,
                    hp3b.reshape(N * 16, LW), src2, dst2)
    h4a, hp4a = _tc_layer_mid(S3[0], h3a, dega, W4, b3)
    h4b, hp4b = _tc_layer_mid(S3[1], h3b, degb, W4, b3)
    S4 = _sc_spmm16(hp4a.reshape(N * 16, LW), src1, dst1,
                    hp4b.reshape(N * 16, LW), src2, dst2)
    xfa = _tc_layer_fin(S4[0], h4a, dega, b4)
    xfb = _tc_layer_fin(S4[1], h4b, degb, b4)

    x1 = _pool(xfa, graph1_batch)
    x2 = _pool(xfb, graph2_batch)
    x12 = x1 - x2
    x21 = -x1 + x2
    x = jnp.concatenate([x12, x21], axis=1)
    out = _head(x, fc1_W, fc1_b, fc2_W, fc2_b, fc3_W, fc3_b)
    return out[:, :1]


# speedup vs baseline: 1.2917x; 1.2917x over previous
import functools

import jax
import jax.numpy as jnp
from jax import lax
from jax.experimental import pallas as pl
from jax.experimental.pallas import tpu as pltpu
from jax.experimental.pallas import tpu_sc as plsc

N = 50000
E = 800000
G = 128

NC = 2
NS = 16
CHD = 1000
CHS = 2000
NP = 50176
RPS = NP // NS
DRN = 784
LW = 16

_mesh = plsc.VectorSubcoreMesh(core_axis_name="c", subcore_axis_name="s")
_UNTILED = pltpu.CompilerParams(use_tc_tiling_on_sc=False)

NB = 1000


@functools.partial(
    pl.kernel, mesh=_mesh,
    out_type=jax.ShapeDtypeStruct((NC, NP, LW), jnp.float32),
    scratch_types=[
        pltpu.VMEM((CHD,), jnp.int32),
        pltpu.VMEM((CHD, LW), jnp.float32),
        pltpu.VMEM((DRN, LW), jnp.float32),
        pltpu.VMEM_SHARED((NP, LW), jnp.float32),
    ],
    compiler_params=_UNTILED,
)
def _sc_degree(dst_hbm, out_hbm, didx_v, ones_v, z_v, acc_sh):
    cid = lax.axis_index("c")
    sid = lax.axis_index("s")
    z_v[...] = jnp.zeros_like(z_v)
    for k in range(RPS // DRN):
        pltpu.sync_copy(z_v, acc_sh.at[pl.ds(sid * RPS + k * DRN, DRN)])
    ones_v[...] = jnp.ones_like(ones_v)
    plsc.subcore_barrier()
    ecore = E // NC
    esub = ecore // NS

    def _chunk(k, carry):
        base = cid * ecore + sid * esub + k * CHD
        pltpu.sync_copy(dst_hbm.at[pl.ds(base, CHD)], didx_v)
        pltpu.sync_copy(ones_v, acc_sh.at[didx_v], add=True)
        return carry

    lax.fori_loop(0, esub // CHD, _chunk, 0)
    plsc.subcore_barrier()
    for k in range(RPS // DRN):
        pltpu.sync_copy(acc_sh.at[pl.ds(sid * RPS + k * DRN, DRN)], z_v)
        pltpu.sync_copy(z_v, out_hbm.at[cid].at[pl.ds(sid * RPS + k * DRN, DRN)])


def _make_sc_spmm(SL):
    @functools.partial(
        pl.kernel, mesh=_mesh,
        out_type=jax.ShapeDtypeStruct((SL, NP, LW), jnp.float32),
        scratch_types=[
            pltpu.VMEM((CHS,), jnp.int32),
            pltpu.VMEM((CHS,), jnp.int32),
            pltpu.VMEM((CHS, LW), jnp.float32),
            pltpu.VMEM((DRN, LW), jnp.float32),
            pltpu.VMEM_SHARED((NP, LW), jnp.float32),
            pltpu.SemaphoreType.DMA,
        ],
        compiler_params=_UNTILED,
    )
    def _sc_spmm(h4_hbm, src_hbm, dst_hbm, out_hbm,
                 sidx_v, didx_v, rows_v, z_v, acc_sh, sem):
        cid = lax.axis_index("c")
        sid = lax.axis_index("s")
        esub = E // NS
        slc = SL // NC
        for p in range(slc):
            c = cid * slc + p
            z_v[...] = jnp.zeros_like(z_v)
            for k in range(RPS // DRN):
                pltpu.sync_copy(z_v, acc_sh.at[pl.ds(sid * RPS + k * DRN, DRN)])
            plsc.subcore_barrier()

            def _chunk(k, carry, c=c):
                base = sid * esub + k * CHS
                pltpu.sync_copy(src_hbm.at[pl.ds(base, CHS)], sidx_v)
                pltpu.sync_copy(dst_hbm.at[pl.ds(base, CHS)], didx_v)
                pltpu.async_copy(h4_hbm.at[c].at[sidx_v], rows_v, sem).wait()
                pltpu.sync_copy(rows_v, acc_sh.at[didx_v], add=True)
                return carry

            lax.fori_loop(0, esub // CHS, _chunk, 0)
            plsc.subcore_barrier()
            for k in range(RPS // DRN):
                pltpu.sync_copy(acc_sh.at[pl.ds(sid * RPS + k * DRN, DRN)], z_v)
                pltpu.sync_copy(z_v, out_hbm.at[c].at[pl.ds(sid * RPS + k * DRN, DRN)])
            plsc.subcore_barrier()

    return _sc_spmm


_sc_spmm8 = _make_sc_spmm(8)
_sc_spmm16 = _make_sc_spmm(16)


def _dinv_from_deg(deg_ref):
    deg = deg_ref[0][:, 0:1] + deg_ref[1][:, 0:1] + 1.0
    return lax.rsqrt(jnp.maximum(deg, 1.0))


def _store_slabs(hp_ref, hs, SL):
    for c in range(SL):
        hp_ref[c] = hs[:, c * LW:(c + 1) * LW]


def _l1_body(x_ref, W_ref, deg_ref, h_ref, hp_ref):
    dinv = _dinv_from_deg(deg_ref)
    h = jnp.dot(x_ref[...], W_ref[...], preferred_element_type=jnp.float32)
    h_ref[...] = h
    _store_slabs(hp_ref, h * dinv, hp_ref.shape[0])


def _mid_body(SL_prev, S_ref, hprev_ref, deg_ref, W_ref, b_ref, h_ref, hp_ref):
    dinv = _dinv_from_deg(deg_ref)
    S = jnp.concatenate([S_ref[c] for c in range(SL_prev)], axis=1)
    x = jax.nn.relu(dinv * S + (dinv * dinv) * hprev_ref[...] + b_ref[...])
    h = jnp.dot(x, W_ref[...], preferred_element_type=jnp.float32)
    h_ref[...] = h
    _store_slabs(hp_ref, h * dinv, hp_ref.shape[0])


def _fin_body(S_ref, hprev_ref, deg_ref, b_ref, x_ref):
    dinv = _dinv_from_deg(deg_ref)
    S = jnp.concatenate([S_ref[c] for c in range(16)], axis=1)
    x_ref[...] = jax.nn.relu(dinv * S + (dinv * dinv) * hprev_ref[...] + b_ref[...])


def _deg_spec():
    return pl.BlockSpec((NC, NB, LW), lambda i: (0, i, 0))


def _full(shape):
    return pl.BlockSpec(shape, lambda i: tuple(0 for _ in shape))


def _hp_spec(SL):
    return pl.BlockSpec((SL, NB, LW), lambda i: (0, i, 0))


def _tc_layer1(x0, W, deg):
    SL = W.shape[1] // LW
    return pl.pallas_call(
        _l1_body,
        grid=(N // NB,),
        in_specs=[
            pl.BlockSpec((NB, x0.shape[1]), lambda i: (i, 0)),
            _full(W.shape),
            _deg_spec(),
        ],
        out_specs=[
            pl.BlockSpec((NB, W.shape[1]), lambda i: (i, 0)),
            _hp_spec(SL),
        ],
        out_shape=[
            jax.ShapeDtypeStruct((N, W.shape[1]), jnp.float32),
            jax.ShapeDtypeStruct((SL, N, LW), jnp.float32),
        ],
    )(x0, W, deg)


def _tc_layer_mid(S, hprev, deg, W, b):
    SLp = S.shape[0]
    SL = W.shape[1] // LW
    return pl.pallas_call(
        functools.partial(_mid_body, SLp),
        grid=(N // NB,),
        in_specs=[
            _hp_spec(SLp),
            pl.BlockSpec((NB, hprev.shape[1]), lambda i: (i, 0)),
            _deg_spec(),
            _full(W.shape),
            _full((1, SLp * LW)),
        ],
        out_specs=[
            pl.BlockSpec((NB, W.shape[1]), lambda i: (i, 0)),
            _hp_spec(SL),
        ],
        out_shape=[
            jax.ShapeDtypeStruct((N, W.shape[1]), jnp.float32),
            jax.ShapeDtypeStruct((SL, N, LW), jnp.float32),
        ],
    )(S, hprev, deg, W, b.reshape(1, -1))


def _tc_layer_fin(S, hprev, deg, b):
    return pl.pallas_call(
        _fin_body,
        grid=(N // NB,),
        in_specs=[
            _hp_spec(16),
            pl.BlockSpec((NB, 256), lambda i: (i, 0)),
            _deg_spec(),
            _full((1, 256)),
        ],
        out_specs=pl.BlockSpec((NB, 256), lambda i: (i, 0)),
        out_shape=jax.ShapeDtypeStruct((N, 256), jnp.float32),
    )(S, hprev, deg, b.reshape(1, -1))


def _head_body(x12_ref, fc1_W_ref, fc1_b_ref, fc2_W_ref, fc2_b_ref,
               fc3_W_ref, fc3_b_ref, out_ref):
    x = x12_ref[...]
    h1 = jax.nn.relu(jnp.dot(x, fc1_W_ref[...], preferred_element_type=jnp.float32) + fc1_b_ref[...])
    h2 = jax.nn.relu(jnp.dot(h1, fc2_W_ref[...], preferred_element_type=jnp.float32) + fc2_b_ref[...])
    h3 = jnp.tanh(jnp.dot(h2, fc3_W_ref[...], preferred_element_type=jnp.float32) + fc3_b_ref[...])
    out_ref[...] = h3


def _head(x, fc1_W, fc1_b, fc2_W, fc2_b, fc3_W, fc3_b):
    return pl.pallas_call(
        _head_body,
        out_shape=jax.ShapeDtypeStruct((G, 128), jnp.float32),
    )(x, fc1_W, fc1_b.reshape(1, -1), fc2_W, fc2_b.reshape(1, -1),
      jnp.pad(fc3_W, ((0, 0), (0, 127))), jnp.pad(fc3_b, (0, 127)).reshape(1, -1))


def _drug_encoder(x, edge_index, batch, params):
    src = edge_index[0]
    dst = edge_index[1]
    deg = _sc_degree(dst)
    (W1, b1, W2, b2, W3, b3, W4, b4) = params

    x0 = jnp.pad(x, ((0, 0), (0, 128 - x.shape[1])))
    W1p = jnp.pad(W1, ((0, 128 - W1.shape[0]), (0, 0)))

    h1, hp1 = _tc_layer1(x0, W1p, deg)
    S1 = _sc_spmm8(hp1, src, dst)
    h2, hp2 = _tc_layer_mid(S1, h1, deg, W2, b1)
    S2 = _sc_spmm16(hp2, src, dst)
    h3, hp3 = _tc_layer_mid(S2, h2, deg, W3, b2)
    S3 = _sc_spmm16(hp3, src, dst)
    h4, hp4 = _tc_layer_mid(S3, h3, deg, W4, b3)
    S4 = _sc_spmm16(hp4, src, dst)
    xf = _tc_layer_fin(S4, h4, deg, b4)

    cnt = jax.ops.segment_sum(jnp.ones((N,), jnp.float32), batch, num_segments=G)
    gmp = jax.ops.segment_max(xf, batch, num_segments=G)
    gmp = jnp.where(cnt[:, None] > 0, gmp, 0.0)
    gap = jax.ops.segment_sum(xf, batch, num_segments=G) / jnp.maximum(cnt, 1.0)[:, None]
    return jnp.concatenate([gmp, gap], axis=1)


def kernel(graph1_x, graph1_edge_index, graph1_batch, graph2_x, graph2_edge_index, graph2_batch, W1, b1, W2, b2, W3, b3, W4, b4, fc1_W, fc1_b, fc2_W, fc2_b, fc3_W, fc3_b):
    params = (W1, b1, W2, b2, W3, b3, W4, b4)
    x1 = _drug_encoder(graph1_x, graph1_edge_index, graph1_batch, params)
    x2 = _drug_encoder(graph2_x, graph2_edge_index, graph2_batch, params)
    x12 = x1 - x2
    x21 = -x1 + x2
    x = jnp.concatenate([x12, x21], axis=1)
    out = _head(x, fc1_W, fc1_b, fc2_W, fc2_b, fc3_W, fc3_b)
    return out[:, :1]


# speedup vs baseline: 1.5238x; 1.1797x over previous
import functools

import jax
import jax.numpy as jnp
from jax import lax
from jax.experimental import pallas as pl
from jax.experimental.pallas import tpu as pltpu
from jax.experimental.pallas import tpu_sc as plsc

N = 50000
E = 800000
G = 128

NC = 2
NS = 16
CHD = 1000
CHS = 1000
NP = 50176
RPS = NP // NS
DRN = 784
LW = 16

_mesh = plsc.VectorSubcoreMesh(core_axis_name="c", subcore_axis_name="s")
_UNTILED = pltpu.CompilerParams(use_tc_tiling_on_sc=False)

NB = 1000


@functools.partial(
    pl.kernel, mesh=_mesh,
    out_type=jax.ShapeDtypeStruct((NC, NP, LW), jnp.float32),
    scratch_types=[
        pltpu.VMEM((CHD,), jnp.int32),
        pltpu.VMEM((CHD, LW), jnp.float32),
        pltpu.VMEM((DRN, LW), jnp.float32),
        pltpu.VMEM_SHARED((NP, LW), jnp.float32),
    ],
    compiler_params=_UNTILED,
)
def _sc_degree(dst_hbm, out_hbm, didx_v, ones_v, z_v, acc_sh):
    cid = lax.axis_index("c")
    sid = lax.axis_index("s")
    z_v[...] = jnp.zeros_like(z_v)
    for k in range(RPS // DRN):
        pltpu.sync_copy(z_v, acc_sh.at[pl.ds(sid * RPS + k * DRN, DRN)])
    ones_v[...] = jnp.ones_like(ones_v)
    plsc.subcore_barrier()
    ecore = E // NC
    esub = ecore // NS

    def _chunk(k, carry):
        base = cid * ecore + sid * esub + k * CHD
        pltpu.sync_copy(dst_hbm.at[pl.ds(base, CHD)], didx_v)
        pltpu.sync_copy(ones_v, acc_sh.at[didx_v], add=True)
        return carry

    lax.fori_loop(0, esub // CHD, _chunk, 0)
    plsc.subcore_barrier()
    for k in range(RPS // DRN):
        pltpu.sync_copy(acc_sh.at[pl.ds(sid * RPS + k * DRN, DRN)], z_v)
        pltpu.sync_copy(z_v, out_hbm.at[cid].at[pl.ds(sid * RPS + k * DRN, DRN)])


def _make_sc_spmm(SL):
    @functools.partial(
        pl.kernel, mesh=_mesh,
        out_type=jax.ShapeDtypeStruct((SL, NP, LW), jnp.float32),
        scratch_types=[
            pltpu.VMEM((CHS,), jnp.int32),
            pltpu.VMEM((CHS,), jnp.int32),
            pltpu.VMEM((CHS, LW), jnp.float32),
            pltpu.VMEM((CHS,), jnp.int32),
            pltpu.VMEM((CHS,), jnp.int32),
            pltpu.VMEM((CHS, LW), jnp.float32),
            pltpu.VMEM((DRN, LW), jnp.float32),
            pltpu.VMEM_SHARED((NP, LW), jnp.float32),
            pltpu.SemaphoreType.DMA,
            pltpu.SemaphoreType.DMA,
        ],
        compiler_params=_UNTILED,
    )
    def _sc_spmm(h4_hbm, src_hbm, dst_hbm, out_hbm,
                 sidx_a, didx_a, rows_a, sidx_b, didx_b, rows_b,
                 z_v, acc_sh, sem_a, sem_b):
        cid = lax.axis_index("c")
        sid = lax.axis_index("s")
        esub = E // NS
        slc = SL // NC
        nch = esub // CHS

        def _load_start(k, sidx_v, didx_v, rows_v, sem, c):
            base = sid * esub + k * CHS
            pltpu.sync_copy(src_hbm.at[pl.ds(base, CHS)], sidx_v)
            pltpu.sync_copy(dst_hbm.at[pl.ds(base, CHS)], didx_v)
            return pltpu.async_copy(h4_hbm.at[c].at[sidx_v], rows_v, sem)

        for p in range(slc):
            c = cid * slc + p
            z_v[...] = jnp.zeros_like(z_v)
            for k in range(RPS // DRN):
                pltpu.sync_copy(z_v, acc_sh.at[pl.ds(sid * RPS + k * DRN, DRN)])
            plsc.subcore_barrier()

            _load_start(0, sidx_a, didx_a, rows_a, sem_a, c)

            def _wait(sidx_v, rows_v, sem, c=c):
                pltpu.make_async_copy(h4_hbm.at[c].at[sidx_v], rows_v, sem).wait()

            def _pair(j, carry, c=c):
                _load_start(2 * j + 1, sidx_b, didx_b, rows_b, sem_b, c)
                _wait(sidx_a, rows_a, sem_a)
                pltpu.sync_copy(rows_a, acc_sh.at[didx_a], add=True)
                _load_start(2 * j + 2, sidx_a, didx_a, rows_a, sem_a, c)
                _wait(sidx_b, rows_b, sem_b)
                pltpu.sync_copy(rows_b, acc_sh.at[didx_b], add=True)
                return carry

            lax.fori_loop(0, nch // 2 - 1, _pair, 0)
            _load_start(nch - 1, sidx_b, didx_b, rows_b, sem_b, c)
            _wait(sidx_a, rows_a, sem_a)
            pltpu.sync_copy(rows_a, acc_sh.at[didx_a], add=True)
            _wait(sidx_b, rows_b, sem_b)
            pltpu.sync_copy(rows_b, acc_sh.at[didx_b], add=True)
            plsc.subcore_barrier()
            for k in range(RPS // DRN):
                pltpu.sync_copy(acc_sh.at[pl.ds(sid * RPS + k * DRN, DRN)], z_v)
                pltpu.sync_copy(z_v, out_hbm.at[c].at[pl.ds(sid * RPS + k * DRN, DRN)])
            plsc.subcore_barrier()

    return _sc_spmm


_sc_spmm8 = _make_sc_spmm(8)
_sc_spmm16 = _make_sc_spmm(16)


def _dinv_from_deg(deg_ref):
    deg = deg_ref[0][:, 0:1] + deg_ref[1][:, 0:1] + 1.0
    return lax.rsqrt(jnp.maximum(deg, 1.0))


def _store_slabs(hp_ref, hs, SL):
    for c in range(SL):
        hp_ref[c] = hs[:, c * LW:(c + 1) * LW]


def _l1_body(x_ref, W_ref, deg_ref, h_ref, hp_ref):
    dinv = _dinv_from_deg(deg_ref)
    h = jnp.dot(x_ref[...], W_ref[...], preferred_element_type=jnp.float32)
    h_ref[...] = h
    _store_slabs(hp_ref, h * dinv, hp_ref.shape[0])


def _mid_body(SL_prev, S_ref, hprev_ref, deg_ref, W_ref, b_ref, h_ref, hp_ref):
    dinv = _dinv_from_deg(deg_ref)
    S = jnp.concatenate([S_ref[c] for c in range(SL_prev)], axis=1)
    x = jax.nn.relu(dinv * S + (dinv * dinv) * hprev_ref[...] + b_ref[...])
    h = jnp.dot(x, W_ref[...], preferred_element_type=jnp.float32)
    h_ref[...] = h
    _store_slabs(hp_ref, h * dinv, hp_ref.shape[0])


def _fin_body(S_ref, hprev_ref, deg_ref, b_ref, x_ref):
    dinv = _dinv_from_deg(deg_ref)
    S = jnp.concatenate([S_ref[c] for c in range(16)], axis=1)
    x_ref[...] = jax.nn.relu(dinv * S + (dinv * dinv) * hprev_ref[...] + b_ref[...])


def _deg_spec():
    return pl.BlockSpec((NC, NB, LW), lambda i: (0, i, 0))


def _full(shape):
    return pl.BlockSpec(shape, lambda i: tuple(0 for _ in shape))


def _hp_spec(SL):
    return pl.BlockSpec((SL, NB, LW), lambda i: (0, i, 0))


def _tc_layer1(x0, W, deg):
    SL = W.shape[1] // LW
    return pl.pallas_call(
        _l1_body,
        grid=(N // NB,),
        in_specs=[
            pl.BlockSpec((NB, x0.shape[1]), lambda i: (i, 0)),
            _full(W.shape),
            _deg_spec(),
        ],
        out_specs=[
            pl.BlockSpec((NB, W.shape[1]), lambda i: (i, 0)),
            _hp_spec(SL),
        ],
        out_shape=[
            jax.ShapeDtypeStruct((N, W.shape[1]), jnp.float32),
            jax.ShapeDtypeStruct((SL, N, LW), jnp.float32),
        ],
    )(x0, W, deg)


def _tc_layer_mid(S, hprev, deg, W, b):
    SLp = S.shape[0]
    SL = W.shape[1] // LW
    return pl.pallas_call(
        functools.partial(_mid_body, SLp),
        grid=(N // NB,),
        in_specs=[
            _hp_spec(SLp),
            pl.BlockSpec((NB, hprev.shape[1]), lambda i: (i, 0)),
            _deg_spec(),
            _full(W.shape),
            _full((1, SLp * LW)),
        ],
        out_specs=[
            pl.BlockSpec((NB, W.shape[1]), lambda i: (i, 0)),
            _hp_spec(SL),
        ],
        out_shape=[
            jax.ShapeDtypeStruct((N, W.shape[1]), jnp.float32),
            jax.ShapeDtypeStruct((SL, N, LW), jnp.float32),
        ],
    )(S, hprev, deg, W, b.reshape(1, -1))


def _tc_layer_fin(S, hprev, deg, b):
    return pl.pallas_call(
        _fin_body,
        grid=(N // NB,),
        in_specs=[
            _hp_spec(16),
            pl.BlockSpec((NB, 256), lambda i: (i, 0)),
            _deg_spec(),
            _full((1, 256)),
        ],
        out_specs=pl.BlockSpec((NB, 256), lambda i: (i, 0)),
        out_shape=jax.ShapeDtypeStruct((N, 256), jnp.float32),
    )(S, hprev, deg, b.reshape(1, -1))


def _head_body(x12_ref, fc1_W_ref, fc1_b_ref, fc2_W_ref, fc2_b_ref,
               fc3_W_ref, fc3_b_ref, out_ref):
    x = x12_ref[...]
    h1 = jax.nn.relu(jnp.dot(x, fc1_W_ref[...], preferred_element_type=jnp.float32) + fc1_b_ref[...])
    h2 = jax.nn.relu(jnp.dot(h1, fc2_W_ref[...], preferred_element_type=jnp.float32) + fc2_b_ref[...])
    h3 = jnp.tanh(jnp.dot(h2, fc3_W_ref[...], preferred_element_type=jnp.float32) + fc3_b_ref[...])
    out_ref[...] = h3


def _head(x, fc1_W, fc1_b, fc2_W, fc2_b, fc3_W, fc3_b):
    return pl.pallas_call(
        _head_body,
        out_shape=jax.ShapeDtypeStruct((G, 128), jnp.float32),
    )(x, fc1_W, fc1_b.reshape(1, -1), fc2_W, fc2_b.reshape(1, -1),
      jnp.pad(fc3_W, ((0, 0), (0, 127))), jnp.pad(fc3_b, (0, 127)).reshape(1, -1))


def _drug_encoder(x, edge_index, batch, params):
    src = edge_index[0]
    dst = edge_index[1]
    deg = _sc_degree(dst)
    (W1, b1, W2, b2, W3, b3, W4, b4) = params

    x0 = jnp.pad(x, ((0, 0), (0, 128 - x.shape[1])))
    W1p = jnp.pad(W1, ((0, 128 - W1.shape[0]), (0, 0)))

    h1, hp1 = _tc_layer1(x0, W1p, deg)
    S1 = _sc_spmm8(hp1, src, dst)
    h2, hp2 = _tc_layer_mid(S1, h1, deg, W2, b1)
    S2 = _sc_spmm16(hp2, src, dst)
    h3, hp3 = _tc_layer_mid(S2, h2, deg, W3, b2)
    S3 = _sc_spmm16(hp3, src, dst)
    h4, hp4 = _tc_layer_mid(S3, h3, deg, W4, b3)
    S4 = _sc_spmm16(hp4, src, dst)
    xf = _tc_layer_fin(S4, h4, deg, b4)

    cnt = jax.ops.segment_sum(jnp.ones((N,), jnp.float32), batch, num_segments=G)
    gmp = jax.ops.segment_max(xf, batch, num_segments=G)
    gmp = jnp.where(cnt[:, None] > 0, gmp, 0.0)
    gap = jax.ops.segment_sum(xf, batch, num_segments=G) / jnp.maximum(cnt, 1.0)[:, None]
    return jnp.concatenate([gmp, gap], axis=1)


def kernel(graph1_x, graph1_edge_index, graph1_batch, graph2_x, graph2_edge_index, graph2_batch, W1, b1, W2, b2, W3, b3, W4, b4, fc1_W, fc1_b, fc2_W, fc2_b, fc3_W, fc3_b):
    params = (W1, b1, W2, b2, W3, b3, W4, b4)
    x1 = _drug_encoder(graph1_x, graph1_edge_index, graph1_batch, params)
    x2 = _drug_encoder(graph2_x, graph2_edge_index, graph2_batch, params)
    x12 = x1 - x2
    x21 = -x1 + x2
    x = jnp.concatenate([x12, x21], axis=1)
    out = _head(x, fc1_W, fc1_b, fc2_W, fc2_b, fc3_W, fc3_b)
    return out[:, :1]
